# baseline (speedup 1.0000x reference)
import jax
import jax.numpy as jnp
from jax.experimental import pallas as pl

_H = 512
_W = 512
_N = 128
_K1 = 21.7
_K2 = 40.0
_BIG = 1.0e4
_LN2 = 0.6931471805599453


def _road_loss_kernel(hd_ref, pred_ref, out_ref):
    hd = hd_ref[:]
    p0 = pred_ref[:, 0:1]
    p1 = pred_ref[:, 1:2]

    hdn = jnp.concatenate([hd[1:, :], hd[511:, :]], axis=0)
    e = jnp.where(hd != hdn, 0.0, _BIG)

    a = e
    b = jnp.concatenate([jnp.full((1, _W), _BIG, jnp.float32),
                         e[:-1, :]], axis=0)
    for s in (1, 2, 4):
        pad = jnp.full((s, _W), _BIG, dtype=jnp.float32)
        a = jnp.minimum(a, jnp.concatenate([a[s:, :], pad], axis=0) + float(s))
        b = jnp.minimum(b, jnp.concatenate([pad, b[:-s, :]], axis=0) + float(s))
    suba = a.reshape(64, 8, _W)[:, 0, :]
    subb = b.reshape(64, 8, _W)[:, 0, :]
    blk8 = (jax.lax.broadcasted_iota(jnp.int32, (64, _W), 0) * 8
            ).astype(jnp.float32)
    svec = jnp.concatenate([(suba + blk8)[1:, :],
                            jnp.full((1, _W), _BIG, jnp.float32)], axis=0)
    tvec = subb - blk8 + 8.0
    for s in (1, 2, 4, 8, 16, 32):
        pads = jnp.full((s, _W), _BIG, dtype=jnp.float32)
        svec = jnp.minimum(svec, jnp.concatenate([svec[s:, :], pads], axis=0))
        tvec = jnp.minimum(tvec, jnp.concatenate([pads, tvec[:-s, :]], axis=0))
    s_full = jnp.broadcast_to(svec[:, None, :], (64, 8, _W)).reshape(_H, _W)
    t_full = jnp.broadcast_to(tvec[:, None, :], (64, 8, _W)).reshape(_H, _W)
    ii = jax.lax.broadcasted_iota(jnp.int32, (_H, _W), 0).astype(jnp.float32)
    m = jnp.minimum(jnp.minimum(a, b),
                    jnp.minimum(s_full - ii, t_full + (ii - 8.0)))
    dopp = m + 1.0
    dsq = dopp * dopp

    lane = jax.lax.broadcasted_iota(jnp.int32, (_N, _H), 1)
    oh0 = (lane == p0).astype(jnp.float32)
    oh1 = (lane == p1).astype(jnp.float32)
    gd = jnp.dot(oh0, dsq, preferred_element_type=jnp.float32)
    gh = jnp.dot(oh0, hd, preferred_element_type=jnp.float32)
    g1 = (1.0 - gh) * gd
    g0 = gh * gd

    oh1m = (lane == p1 - 1).astype(jnp.float32)
    oh0m = (lane == p0 - 1).astype(jnp.float32)
    gp = jnp.dot(oh1 + oh1m, hd, preferred_element_type=jnp.float32)
    nbr = jnp.sum(gp * (oh0 + oh0m), axis=1, keepdims=True)

    bb = (lane.astype(jnp.float32) - p1.astype(jnp.float32)) ** 2
    dmin1sq = jnp.min(g1 + bb, axis=1, keepdims=True)
    dmin0sq = jnp.min(g0 + bb, axis=1, keepdims=True)

    outside_frame = (p0 < 0) | (p0 > _H) | (p1 < 0) | (p1 > _W)
    valid = (p0 >= 1) & (p1 >= 1)
    outside_road = valid & (nbr > 0.5)
    loss_out = jnp.exp(jnp.sqrt(dmin0sq) * (_LN2 / _K2))
    loss_in = jnp.exp(-dmin1sq * (1.0 / _K1))
    per = jnp.where(outside_frame, 0.0,
                    jnp.where(outside_road, loss_out, loss_in))
    out_ref[:, :] = jnp.sum(per, axis=0, keepdims=True) * (1.0 / _N)


@jax.jit
def _run(hd_map, prediction):
    return pl.pallas_call(
        _road_loss_kernel,
        out_shape=jax.ShapeDtypeStruct((1, 1), jnp.float32),
    )(hd_map, prediction)


def kernel(hd_map, prediction):
    out = _run(hd_map, prediction)
    return out[0, 0]


# speedup vs baseline: 1.1639x; 1.1639x over previous
import jax
import jax.numpy as jnp
from jax.experimental import pallas as pl

_H = 512
_W = 512
_N = 128
_K1 = 21.7
_K2 = 40.0
_BIG = 1.0e4
_LN2 = 0.6931471805599453


def _road_loss_kernel(hd_ref, pred_ref, out_ref):
    hd = hd_ref[:]
    p0 = pred_ref[:, 0:1]
    p1 = pred_ref[:, 1:2]

    hdn = jnp.concatenate([hd[1:, :], hd[511:, :]], axis=0)
    e = jnp.where(hd != hdn, 0.0, _BIG)

    a = e
    b = jnp.concatenate([jnp.full((1, _W), _BIG, jnp.float32),
                         e[:-1, :]], axis=0)
    for s in (1, 2, 4):
        pad = jnp.full((s, _W), _BIG, dtype=jnp.float32)
        a = jnp.minimum(a, jnp.concatenate([a[s:, :], pad], axis=0) + float(s))
        b = jnp.minimum(b, jnp.concatenate([pad, b[:-s, :]], axis=0) + float(s))
    sub_sel = (jax.lax.broadcasted_iota(jnp.int32, (64, _H), 1)
               == jax.lax.broadcasted_iota(jnp.int32, (64, _H), 0) * 8
               ).astype(jnp.float32)
    suba = jnp.dot(sub_sel, a, preferred_element_type=jnp.float32)
    subb = jnp.dot(sub_sel, b, preferred_element_type=jnp.float32)
    blk8 = (jax.lax.broadcasted_iota(jnp.int32, (64, _W), 0) * 8
            ).astype(jnp.float32)
    svec = jnp.concatenate([(suba + blk8)[1:, :],
                            jnp.full((1, _W), _BIG, jnp.float32)], axis=0)
    tvec = subb - blk8 + 8.0
    for s in (1, 2, 4, 8, 16, 32):
        pads = jnp.full((s, _W), _BIG, dtype=jnp.float32)
        svec = jnp.minimum(svec, jnp.concatenate([svec[s:, :], pads], axis=0))
        tvec = jnp.minimum(tvec, jnp.concatenate([pads, tvec[:-s, :]], axis=0))
    bcast = (jax.lax.broadcasted_iota(jnp.int32, (_H, 64), 1)
             == jax.lax.broadcasted_iota(jnp.int32, (_H, 64), 0) // 8
             ).astype(jnp.float32)
    s_full = jnp.dot(bcast, svec, preferred_element_type=jnp.float32)
    t_full = jnp.dot(bcast, tvec, preferred_element_type=jnp.float32)
    ii = jax.lax.broadcasted_iota(jnp.int32, (_H, _W), 0).astype(jnp.float32)
    m = jnp.minimum(jnp.minimum(a, b),
                    jnp.minimum(s_full - ii, t_full + (ii - 8.0)))
    dopp = m + 1.0
    dsq = dopp * dopp

    lane = jax.lax.broadcasted_iota(jnp.int32, (_N, _H), 1)
    oh0 = (lane == p0).astype(jnp.float32)
    oh1 = (lane == p1).astype(jnp.float32)
    gd = jnp.dot(oh0, dsq, preferred_element_type=jnp.float32)
    gh = jnp.dot(oh0, hd, preferred_element_type=jnp.float32)
    g1 = (1.0 - gh) * gd
    g0 = gh * gd

    oh1m = (lane == p1 - 1).astype(jnp.float32)
    oh0m = (lane == p0 - 1).astype(jnp.float32)
    gp = jnp.dot(oh1 + oh1m, hd, preferred_element_type=jnp.float32)
    nbr = jnp.sum(gp * (oh0 + oh0m), axis=1, keepdims=True)

    bb = (lane.astype(jnp.float32) - p1.astype(jnp.float32)) ** 2
    dmin1sq = jnp.min(g1 + bb, axis=1, keepdims=True)
    dmin0sq = jnp.min(g0 + bb, axis=1, keepdims=True)

    outside_frame = (p0 < 0) | (p0 > _H) | (p1 < 0) | (p1 > _W)
    valid = (p0 >= 1) & (p1 >= 1)
    outside_road = valid & (nbr > 0.5)
    loss_out = jnp.exp(jnp.sqrt(dmin0sq) * (_LN2 / _K2))
    loss_in = jnp.exp(-dmin1sq * (1.0 / _K1))
    per = jnp.where(outside_frame, 0.0,
                    jnp.where(outside_road, loss_out, loss_in))
    out_ref[:, :] = jnp.sum(per, axis=0, keepdims=True) * (1.0 / _N)


@jax.jit
def _run(hd_map, prediction):
    return pl.pallas_call(
        _road_loss_kernel,
        out_shape=jax.ShapeDtypeStruct((1, 1), jnp.float32),
    )(hd_map, prediction)


def kernel(hd_map, prediction):
    out = _run(hd_map, prediction)
    return out[0, 0]


# speedup vs baseline: 1.1968x; 1.0283x over previous
import jax
import jax.numpy as jnp
from jax.experimental import pallas as pl

_H = 512
_W = 512
_N = 128
_K1 = 21.7
_K2 = 40.0
_BIG = 1.0e4
_LN2 = 0.6931471805599453


def _road_loss_kernel(hd_ref, pred_ref, out_ref):
    hd = hd_ref[:]
    p0 = pred_ref[:, 0:1]
    p1 = pred_ref[:, 1:2]

    hdn = jnp.concatenate([hd[1:, :], hd[511:, :]], axis=0)
    e = jnp.where(hd != hdn, 0.0, _BIG)

    em = jnp.minimum(e, jnp.concatenate(
        [jnp.full((1, _W), _BIG, jnp.float32), e[:-1, :]], axis=0))
    a = em
    for s in (1, 2, 4):
        pad = jnp.full((s, _W), _BIG, dtype=jnp.float32)
        up = jnp.concatenate([a[s:, :], pad], axis=0)
        dn = jnp.concatenate([pad, a[:-s, :]], axis=0)
        a = jnp.minimum(a, jnp.minimum(up, dn) + float(s))
    sub8 = jax.lax.broadcasted_iota(jnp.int32, (128, _H), 0)
    lane8 = jax.lax.broadcasted_iota(jnp.int32, (128, _H), 1)
    sub_sel = ((lane8 == sub8 * 8)
               | (lane8 == (sub8 - 64) * 8 + 7)).astype(jnp.float32)
    fg = jnp.dot(sub_sel, a, preferred_element_type=jnp.float32)
    blk8 = (jax.lax.broadcasted_iota(jnp.int32, (64, _W), 0) * 8
            ).astype(jnp.float32)
    svec = jnp.concatenate([(fg[:64, :] + blk8)[1:, :],
                            jnp.full((1, _W), _BIG, jnp.float32)], axis=0)
    tvec = jnp.concatenate([jnp.full((1, _W), _BIG, jnp.float32),
                            (fg[64:, :] - blk8)[:-1, :] - 7.0], axis=0)
    for s in (1, 2, 4, 8, 16, 32):
        pads = jnp.full((s, _W), _BIG, dtype=jnp.float32)
        svec = jnp.minimum(svec, jnp.concatenate([svec[s:, :], pads], axis=0))
        tvec = jnp.minimum(tvec, jnp.concatenate([pads, tvec[:-s, :]], axis=0))
    bcast = (jax.lax.broadcasted_iota(jnp.int32, (_H, 64), 1)
             == jax.lax.broadcasted_iota(jnp.int32, (_H, 64), 0) // 8
             ).astype(jnp.float32)
    s_full = jnp.dot(bcast, svec, preferred_element_type=jnp.float32)
    t_full = jnp.dot(bcast, tvec, preferred_element_type=jnp.float32)
    ii = jax.lax.broadcasted_iota(jnp.int32, (_H, _W), 0).astype(jnp.float32)
    m = jnp.minimum(a, jnp.minimum(s_full - ii, t_full + ii))
    dopp = m + 1.0
    dsq = dopp * dopp

    lane = jax.lax.broadcasted_iota(jnp.int32, (_N, _H), 1)
    oh0 = (lane == p0).astype(jnp.float32)
    oh1 = (lane == p1).astype(jnp.float32)
    gd = jnp.dot(oh0, dsq, preferred_element_type=jnp.float32)
    gh = jnp.dot(oh0, hd, preferred_element_type=jnp.float32)
    g1 = (1.0 - gh) * gd
    g0 = gh * gd

    oh1m = (lane == p1 - 1).astype(jnp.float32)
    oh0m = (lane == p0 - 1).astype(jnp.float32)
    gp = jnp.dot(oh1 + oh1m, hd, preferred_element_type=jnp.float32)
    nbr = jnp.sum(gp * (oh0 + oh0m), axis=1, keepdims=True)

    bb = (lane.astype(jnp.float32) - p1.astype(jnp.float32)) ** 2
    dmin1sq = jnp.min(g1 + bb, axis=1, keepdims=True)
    dmin0sq = jnp.min(g0 + bb, axis=1, keepdims=True)

    outside_frame = (p0 < 0) | (p0 > _H) | (p1 < 0) | (p1 > _W)
    valid = (p0 >= 1) & (p1 >= 1)
    outside_road = valid & (nbr > 0.5)
    loss_out = jnp.exp(jnp.sqrt(dmin0sq) * (_LN2 / _K2))
    loss_in = jnp.exp(-dmin1sq * (1.0 / _K1))
    per = jnp.where(outside_frame, 0.0,
                    jnp.where(outside_road, loss_out, loss_in))
    out_ref[:, :] = jnp.sum(per, axis=0, keepdims=True) * (1.0 / _N)


@jax.jit
def _run(hd_map, prediction):
    return pl.pallas_call(
        _road_loss_kernel,
        out_shape=jax.ShapeDtypeStruct((1, 1), jnp.float32),
    )(hd_map, prediction)


def kernel(hd_map, prediction):
    out = _run(hd_map, prediction)
    return out[0, 0]


# speedup vs baseline: 1.2707x; 1.0617x over previous
import jax
import jax.numpy as jnp
from jax.experimental import pallas as pl

_H = 512
_W = 512
_N = 128
_K1 = 21.7
_K2 = 40.0
_BIG = 1.0e4
_LN2 = 0.6931471805599453


def _road_loss_kernel(hd_ref, pred_ref, out_ref):
    hd = hd_ref[:]
    p0 = pred_ref[:, 0:1]
    p1 = pred_ref[:, 1:2]

    hdn = jnp.concatenate([hd[1:, :], hd[511:, :]], axis=0)
    e = jnp.where(hd != hdn, 0.0, _BIG)

    em = jnp.minimum(e, jnp.concatenate(
        [jnp.full((1, _W), _BIG, jnp.float32), e[:-1, :]], axis=0))
    a = em
    for s in (1, 2, 4):
        pad = jnp.full((s, _W), _BIG, dtype=jnp.float32)
        up = jnp.concatenate([a[s:, :], pad], axis=0)
        dn = jnp.concatenate([pad, a[:-s, :]], axis=0)
        a = jnp.minimum(a, jnp.minimum(up, dn) + float(s))
    sub8 = jax.lax.broadcasted_iota(jnp.int32, (128, _H), 0)
    lane8 = jax.lax.broadcasted_iota(jnp.int32, (128, _H), 1)
    sub_sel = ((lane8 == sub8 * 8)
               | (lane8 == (sub8 - 64) * 8 + 7)).astype(jnp.float32)
    lane = jax.lax.broadcasted_iota(jnp.int32, (_N, _H), 1)
    oh0 = (lane == p0).astype(jnp.float32)
    fga = jnp.dot(jnp.concatenate([sub_sel, oh0], axis=0), a,
                  preferred_element_type=jnp.float32)
    fg = fga[:128, :]
    ga = fga[128:, :]
    blk8 = (jax.lax.broadcasted_iota(jnp.int32, (64, _W), 0) * 8
            ).astype(jnp.float32)
    svec = jnp.concatenate([(fg[:64, :] + blk8)[1:, :],
                            jnp.full((1, _W), _BIG, jnp.float32)], axis=0)
    tvec = jnp.concatenate([jnp.full((1, _W), _BIG, jnp.float32),
                            (fg[64:, :] - blk8)[:-1, :] - 7.0], axis=0)
    for s in (1, 2, 4, 8, 16, 32):
        pads = jnp.full((s, _W), _BIG, dtype=jnp.float32)
        svec = jnp.minimum(svec, jnp.concatenate([svec[s:, :], pads], axis=0))
        tvec = jnp.minimum(tvec, jnp.concatenate([pads, tvec[:-s, :]], axis=0))
    lane64 = jax.lax.broadcasted_iota(jnp.int32, (_N, 64), 1)
    ohb = (lane64 == jax.lax.shift_right_logical(p0, 3)).astype(jnp.float32)
    gst = jnp.dot(ohb, jnp.concatenate([svec, tvec], axis=1),
                  preferred_element_type=jnp.float32)
    p0f = p0.astype(jnp.float32)
    gm = jnp.minimum(ga, jnp.minimum(gst[:, :_W] - p0f, gst[:, _W:] + p0f))
    gdo = gm + 1.0
    gd = gdo * gdo
    oh1 = (lane == p1).astype(jnp.float32)
    gh = jnp.dot(oh0, hd, preferred_element_type=jnp.float32)
    g1 = (1.0 - gh) * gd
    g0 = gh * gd

    oh1m = (lane == p1 - 1).astype(jnp.float32)
    oh0m = (lane == p0 - 1).astype(jnp.float32)
    gp = jnp.dot(oh1 + oh1m, hd, preferred_element_type=jnp.float32)
    nbr = jnp.sum(gp * (oh0 + oh0m), axis=1, keepdims=True)

    bb = (lane.astype(jnp.float32) - p1.astype(jnp.float32)) ** 2
    dmin1sq = jnp.min(g1 + bb, axis=1, keepdims=True)
    dmin0sq = jnp.min(g0 + bb, axis=1, keepdims=True)

    outside_frame = (p0 < 0) | (p0 > _H) | (p1 < 0) | (p1 > _W)
    valid = (p0 >= 1) & (p1 >= 1)
    outside_road = valid & (nbr > 0.5)
    loss_out = jnp.exp(jnp.sqrt(dmin0sq) * (_LN2 / _K2))
    loss_in = jnp.exp(-dmin1sq * (1.0 / _K1))
    per = jnp.where(outside_frame, 0.0,
                    jnp.where(outside_road, loss_out, loss_in))
    out_ref[:, :] = jnp.sum(per, axis=0, keepdims=True) * (1.0 / _N)


@jax.jit
def _run(hd_map, prediction):
    return pl.pallas_call(
        _road_loss_kernel,
        out_shape=jax.ShapeDtypeStruct((1, 1), jnp.float32),
    )(hd_map, prediction)


def kernel(hd_map, prediction):
    out = _run(hd_map, prediction)
    return out[0, 0]


# speedup vs baseline: 1.2806x; 1.0077x over previous
import jax
import jax.numpy as jnp
from jax.experimental import pallas as pl

_H = 512
_W = 512
_N = 128
_K1 = 21.7
_K2 = 40.0
_BIG = 1.0e4
_LN2 = 0.6931471805599453


def _road_loss_kernel(hd_ref, pred_ref, out_ref):
    hd = hd_ref[:]
    p0 = pred_ref[:, 0:1]
    p1 = pred_ref[:, 1:2]

    hdn = jnp.concatenate([hd[1:, :], hd[511:, :]], axis=0)
    e = jnp.where(hd != hdn, 0.0, _BIG)

    em = jnp.minimum(e, jnp.concatenate(
        [jnp.full((1, _W), _BIG, jnp.float32), e[:-1, :]], axis=0))
    a = em
    for s in (1, 2):
        pad = jnp.full((s, _W), _BIG, dtype=jnp.float32)
        up = jnp.concatenate([a[s:, :], pad], axis=0)
        dn = jnp.concatenate([pad, a[:-s, :]], axis=0)
        a = jnp.minimum(a, jnp.minimum(up, dn) + float(s))
    sub4 = jax.lax.broadcasted_iota(jnp.int32, (256, _H), 0)
    lane4 = jax.lax.broadcasted_iota(jnp.int32, (256, _H), 1)
    sub_sel = ((lane4 == sub4 * 4)
               | (lane4 == (sub4 - 128) * 4 + 3)).astype(jnp.float32)
    lane = jax.lax.broadcasted_iota(jnp.int32, (_N, _H), 1)
    oh0 = (lane == p0).astype(jnp.float32)
    fga = jnp.dot(jnp.concatenate([sub_sel, oh0], axis=0), a,
                  preferred_element_type=jnp.float32)
    fg = fga[:256, :]
    ga = fga[256:, :]
    blk4 = (jax.lax.broadcasted_iota(jnp.int32, (128, _W), 0) * 4
            ).astype(jnp.float32)
    svec = jnp.concatenate([(fg[:128, :] + blk4)[1:, :],
                            jnp.full((1, _W), _BIG, jnp.float32)], axis=0)
    tvec = jnp.concatenate([jnp.full((1, _W), _BIG, jnp.float32),
                            (fg[128:, :] - blk4)[:-1, :] - 3.0], axis=0)
    for s in (1, 2, 4, 8, 16, 32, 64):
        pads = jnp.full((s, _W), _BIG, dtype=jnp.float32)
        svec = jnp.minimum(svec, jnp.concatenate([svec[s:, :], pads], axis=0))
        tvec = jnp.minimum(tvec, jnp.concatenate([pads, tvec[:-s, :]], axis=0))
    lane128 = jax.lax.broadcasted_iota(jnp.int32, (_N, 128), 1)
    ohb = (lane128 == jax.lax.shift_right_logical(p0, 2)).astype(jnp.float32)
    gst = jnp.dot(ohb, jnp.concatenate([svec, tvec], axis=1),
                  preferred_element_type=jnp.float32)
    p0f = p0.astype(jnp.float32)
    gm = jnp.minimum(ga, jnp.minimum(gst[:, :_W] - p0f, gst[:, _W:] + p0f))
    gdo = gm + 1.0
    gd = gdo * gdo
    oh1 = (lane == p1).astype(jnp.float32)
    gh = jnp.dot(oh0, hd, preferred_element_type=jnp.float32)
    g1 = (1.0 - gh) * gd
    g0 = gh * gd

    oh1m = (lane == p1 - 1).astype(jnp.float32)
    oh0m = (lane == p0 - 1).astype(jnp.float32)
    gp = jnp.dot(oh1 + oh1m, hd, preferred_element_type=jnp.float32)
    nbr = jnp.sum(gp * (oh0 + oh0m), axis=1, keepdims=True)

    bb = (lane.astype(jnp.float32) - p1.astype(jnp.float32)) ** 2
    dmin1sq = jnp.min(g1 + bb, axis=1, keepdims=True)
    dmin0sq = jnp.min(g0 + bb, axis=1, keepdims=True)

    outside_frame = (p0 < 0) | (p0 > _H) | (p1 < 0) | (p1 > _W)
    valid = (p0 >= 1) & (p1 >= 1)
    outside_road = valid & (nbr > 0.5)
    loss_out = jnp.exp(jnp.sqrt(dmin0sq) * (_LN2 / _K2))
    loss_in = jnp.exp(-dmin1sq * (1.0 / _K1))
    per = jnp.where(outside_frame, 0.0,
                    jnp.where(outside_road, loss_out, loss_in))
    out_ref[:, :] = jnp.sum(per, axis=0, keepdims=True) * (1.0 / _N)


@jax.jit
def _run(hd_map, prediction):
    return pl.pallas_call(
        _road_loss_kernel,
        out_shape=jax.ShapeDtypeStruct((1, 1), jnp.float32),
    )(hd_map, prediction)


def kernel(hd_map, prediction):
    out = _run(hd_map, prediction)
    return out[0, 0]
